# baseline (speedup 1.0000x reference)
import functools

import jax
import jax.numpy as jnp
from jax import lax
from jax.experimental import pallas as pl
from jax.experimental.pallas import tpu as pltpu
from jax.experimental.pallas import tpu_sc as plsc

VOCAB = 100000
EMBED_DIM = 128
NUM_CORES = 2
NUM_SUBCORES = 16
NUM_WORKERS = NUM_CORES * NUM_SUBCORES
CHUNK = 200
NBUF = 4
N_SC = 691200
BLK_TC = 4000


def _gather_sc(table, flat_ids, n_out):
    n_sc = flat_ids.shape[0]
    per_worker = n_sc // NUM_WORKERS
    nchunks = per_worker // CHUNK
    assert per_worker % CHUNK == 0 and nchunks % NBUF == 0
    mesh = plsc.VectorSubcoreMesh(core_axis_name="c", subcore_axis_name="s")

    @functools.partial(
        pl.kernel,
        mesh=mesh,
        out_type=jax.ShapeDtypeStruct((n_out, EMBED_DIM), table.dtype),
        scratch_types=[
            pltpu.VMEM((per_worker,), jnp.int32),
            pltpu.VMEM((NBUF, CHUNK, EMBED_DIM), jnp.float32),
            pltpu.SemaphoreType.DMA((NBUF,)),
            pltpu.SemaphoreType.DMA((NBUF,)),
        ],
    )
    def gather_kernel(table_hbm, ids_hbm, out_hbm, idx_v, bufs, gsems, wsems):
        wid = lax.axis_index("s") * NUM_CORES + lax.axis_index("c")
        base = wid * per_worker
        pltpu.sync_copy(ids_hbm.at[pl.ds(base, per_worker)], idx_v)

        def out_row(c):
            return (c * NUM_WORKERS + wid) * CHUNK

        def start_gather(c, b):
            pltpu.async_copy(
                table_hbm.at[idx_v.at[pl.ds(c * CHUNK, CHUNK)]],
                bufs.at[b], gsems.at[b])

        def wait_gather(c, b):
            pltpu.make_async_copy(
                table_hbm.at[idx_v.at[pl.ds(c * CHUNK, CHUNK)]],
                bufs.at[b], gsems.at[b]).wait()

        def start_write(c, b):
            pltpu.async_copy(
                bufs.at[b], out_hbm.at[pl.ds(out_row(c), CHUNK)],
                wsems.at[b])

        def wait_write(c, b):
            pltpu.make_async_copy(
                bufs.at[b], out_hbm.at[pl.ds(out_row(c), CHUNK)],
                wsems.at[b]).wait()

        for b in range(NBUF):
            start_gather(b, b)

        @pl.loop(0, nchunks, step=NBUF)
        def _(g):
            for b in range(NBUF):
                wait_gather(g + b, b)
                start_write(g + b, b)
            for b in range(NBUF):
                @pl.when(g + b + NBUF < nchunks)
                def _():
                    wait_write(g + b, b)
                    start_gather(g + b + NBUF, b)

        for b in range(NBUF):
            wait_write(nchunks - NBUF + b, b)

    return gather_kernel(table, flat_ids)


def _gather_tc(table, ids):
    n = ids.shape[0]
    grid = (n // BLK_TC,)
    ids3 = ids.reshape(n // BLK_TC, 1, BLK_TC)

    def body(ids_ref, table_ref, out_ref):
        def row(r, carry):
            out_ref[r, :] = table_ref[ids_ref[0, 0, r], :]
            return carry

        lax.fori_loop(0, BLK_TC, row, 0, unroll=8)

    return pl.pallas_call(
        body,
        grid=grid,
        in_specs=[
            pl.BlockSpec((1, 1, BLK_TC), lambda i: (i, 0, 0),
                         memory_space=pltpu.SMEM),
            pl.BlockSpec((VOCAB, EMBED_DIM), lambda i: (0, 0)),
        ],
        out_specs=pl.BlockSpec((BLK_TC, EMBED_DIM), lambda i: (i, 0)),
        out_shape=jax.ShapeDtypeStruct((n, EMBED_DIM), table.dtype),
    )(ids3, table)


def kernel(token_ids, table):
    batch, seq = token_ids.shape
    n = batch * seq
    flat = token_ids.reshape(n).astype(jnp.int32)

    ids_sc = flat[:N_SC]
    nchunks = N_SC // (NUM_WORKERS * CHUNK)
    perm = ids_sc.reshape(nchunks, NUM_WORKERS, CHUNK).transpose(1, 0, 2)
    out = _gather_sc(table, perm.reshape(N_SC), n)

    out_tc = _gather_tc(table, flat[N_SC:])
    out = lax.dynamic_update_slice(out, out_tc, (N_SC, 0))
    return out.reshape(batch, seq, EMBED_DIM)


# speedup vs baseline: 1.1002x; 1.1002x over previous
import functools

import jax
import jax.numpy as jnp
from jax import lax
from jax.experimental import pallas as pl
from jax.experimental.pallas import tpu as pltpu
from jax.experimental.pallas import tpu_sc as plsc

EMBED_DIM = 128
NUM_CORES = 2
NUM_SUBCORES = 16
NUM_WORKERS = NUM_CORES * NUM_SUBCORES
CHUNK = 200
NBUF = 4


def _gather_sc(table, flat_ids, n_out):
    n_sc = flat_ids.shape[0]
    per_worker = n_sc // NUM_WORKERS
    nchunks = per_worker // CHUNK
    assert per_worker % CHUNK == 0 and nchunks % NBUF == 0
    mesh = plsc.VectorSubcoreMesh(core_axis_name="c", subcore_axis_name="s")

    @functools.partial(
        pl.kernel,
        mesh=mesh,
        out_type=jax.ShapeDtypeStruct((n_out, EMBED_DIM), table.dtype),
        scratch_types=[
            pltpu.VMEM((per_worker,), jnp.int32),
            pltpu.VMEM((NBUF, CHUNK, EMBED_DIM), jnp.float32),
            pltpu.SemaphoreType.DMA((NBUF,)),
            pltpu.SemaphoreType.DMA((NBUF,)),
        ],
    )
    def gather_kernel(table_hbm, ids_hbm, out_hbm, idx_v, bufs, gsems, wsems):
        wid = lax.axis_index("s") * NUM_CORES + lax.axis_index("c")
        base = wid * per_worker
        pltpu.sync_copy(ids_hbm.at[pl.ds(base, per_worker)], idx_v)

        def out_row(c):
            return (c * NUM_WORKERS + wid) * CHUNK

        def start_gather(c, b):
            pltpu.async_copy(
                table_hbm.at[idx_v.at[pl.ds(c * CHUNK, CHUNK)]],
                bufs.at[b], gsems.at[b])

        def wait_gather(c, b):
            pltpu.make_async_copy(
                table_hbm.at[idx_v.at[pl.ds(c * CHUNK, CHUNK)]],
                bufs.at[b], gsems.at[b]).wait()

        def start_write(c, b):
            pltpu.async_copy(
                bufs.at[b], out_hbm.at[pl.ds(out_row(c), CHUNK)],
                wsems.at[b])

        def wait_write(c, b):
            pltpu.make_async_copy(
                bufs.at[b], out_hbm.at[pl.ds(out_row(c), CHUNK)],
                wsems.at[b]).wait()

        for b in range(NBUF):
            start_gather(b, b)

        @pl.loop(0, nchunks, step=NBUF)
        def _(g):
            for b in range(NBUF):
                wait_gather(g + b, b)
                start_write(g + b, b)
            for b in range(NBUF):
                @pl.when(g + b + NBUF < nchunks)
                def _():
                    wait_write(g + b, b)
                    start_gather(g + b + NBUF, b)

        for b in range(NBUF):
            wait_write(nchunks - NBUF + b, b)

    return gather_kernel(table, flat_ids)


def kernel(token_ids, table):
    batch, seq = token_ids.shape
    n = batch * seq
    flat = token_ids.reshape(n).astype(jnp.int32)
    nchunks = n // (NUM_WORKERS * CHUNK)
    perm = flat.reshape(nchunks, NUM_WORKERS, CHUNK).transpose(1, 0, 2)
    out = _gather_sc(table, perm.reshape(n), n)
    return out.reshape(batch, seq, EMBED_DIM)


# speedup vs baseline: 1.1328x; 1.0296x over previous
import functools

import jax
import jax.numpy as jnp
from jax import lax
from jax.experimental import pallas as pl
from jax.experimental.pallas import tpu as pltpu
from jax.experimental.pallas import tpu_sc as plsc

EMBED_DIM = 128
NUM_CORES = 2
NUM_SUBCORES = 16
NUM_WORKERS = NUM_CORES * NUM_SUBCORES
CHUNK = 200
NBUF = 4


def _gather_sc(table, flat_ids):
    n = flat_ids.shape[0]
    per_worker = n // NUM_WORKERS
    nchunks = per_worker // CHUNK
    assert per_worker % CHUNK == 0 and nchunks % NBUF == 0
    mesh = plsc.VectorSubcoreMesh(core_axis_name="c", subcore_axis_name="s")

    @functools.partial(
        pl.kernel,
        mesh=mesh,
        out_type=jax.ShapeDtypeStruct((n, EMBED_DIM), table.dtype),
        scratch_types=(
            [pltpu.VMEM((CHUNK,), jnp.int32) for _ in range(NBUF)]
            + [
                pltpu.VMEM((NBUF, CHUNK, EMBED_DIM), jnp.float32),
                pltpu.SemaphoreType.DMA((NBUF,)),
                pltpu.SemaphoreType.DMA((NBUF,)),
                pltpu.SemaphoreType.DMA((NBUF,)),
            ]
        ),
    )
    def gather_kernel(table_hbm, ids_hbm, out_hbm, ib0, ib1, ib2, ib3, bufs,
                      isems, gsems, wsems):
        ibufs = [ib0, ib1, ib2, ib3]
        wid = lax.axis_index("s") * NUM_CORES + lax.axis_index("c")

        def row0(c):
            return (c * NUM_WORKERS + wid) * CHUNK

        def start_idx(c, b):
            pltpu.async_copy(ids_hbm.at[pl.ds(row0(c), CHUNK)],
                             ibufs[b], isems.at[b])

        def wait_idx(c, b):
            pltpu.make_async_copy(ids_hbm.at[pl.ds(row0(c), CHUNK)],
                                  ibufs[b], isems.at[b]).wait()

        def start_gather(b):
            pltpu.async_copy(table_hbm.at[ibufs[b]], bufs.at[b],
                             gsems.at[b])

        def wait_gather(b):
            pltpu.make_async_copy(table_hbm.at[ibufs[b]], bufs.at[b],
                                  gsems.at[b]).wait()

        def start_write(c, b):
            pltpu.async_copy(bufs.at[b], out_hbm.at[pl.ds(row0(c), CHUNK)],
                             wsems.at[b])

        def wait_write(c, b):
            pltpu.make_async_copy(bufs.at[b],
                                  out_hbm.at[pl.ds(row0(c), CHUNK)],
                                  wsems.at[b]).wait()

        for b in range(NBUF):
            start_idx(b, b)
        for b in range(NBUF):
            wait_idx(b, b)
            start_gather(b)

        @pl.loop(0, nchunks, step=NBUF)
        def _(g):
            for b in range(NBUF):
                wait_gather(b)

                @pl.when(g + b + NBUF < nchunks)
                def _():
                    start_idx(g + b + NBUF, b)

                start_write(g + b, b)
            for b in range(NBUF):
                @pl.when(g + b + NBUF < nchunks)
                def _():
                    wait_write(g + b, b)
                    wait_idx(g + b + NBUF, b)
                    start_gather(b)

        for b in range(NBUF):
            wait_write(nchunks - NBUF + b, b)

    return gather_kernel(table, flat_ids)


def kernel(token_ids, table):
    batch, seq = token_ids.shape
    flat = token_ids.reshape(batch * seq).astype(jnp.int32)
    out = _gather_sc(table, flat)
    return out.reshape(batch, seq, EMBED_DIM)
